# baseline (speedup 1.0000x reference)
import functools

import jax
import jax.numpy as jnp
from jax import lax
from jax.experimental import pallas as pl
from jax.experimental.pallas import tpu as pltpu
from jax.experimental.pallas import tpu_sc as plsc

_NW = 32
_B = 128


def _newton_rsqrt(d):
    i = lax.bitcast_convert_type(d, jnp.int32)
    i = jnp.int32(0x5F3759DF) - (i >> 1)
    y = lax.bitcast_convert_type(i, jnp.float32)
    for _ in range(3):
        y = y * (1.5 - 0.5 * d * y * y)
    return jnp.where(d > 0, y, 0.0)


def _sc_kernel(n_pad, kb, kb0, kb1):
    st = n_pad // 16
    ncp = st // _B
    kbm = max(kb0, kb1)
    mesh = plsc.VectorSubcoreMesh(core_axis_name="c", subcore_axis_name="s")

    @functools.partial(
        pl.kernel, mesh=mesh,
        compiler_params=pltpu.CompilerParams(use_tc_tiling_on_sc=False),
        out_type=[
            jax.ShapeDtypeStruct((2, n_pad, 64), jnp.float32),
            jax.ShapeDtypeStruct((8, n_pad), jnp.float32),
        ],
        scratch_types=[
            pltpu.VMEM((kbm, _B), jnp.int32),
            pltpu.VMEM((kbm, _B), jnp.int32),
            pltpu.VMEM((kbm, _B), jnp.float32),
            pltpu.VMEM((_B,), jnp.float32),
            pltpu.VMEM((_B, 128), jnp.float32),
            pltpu.VMEM((_B, 128), jnp.float32),
            pltpu.VMEM((_B, 64), jnp.float32),
            pltpu.VMEM((_B, 64), jnp.float32),
            pltpu.VMEM((_B, 64), jnp.float32),
            pltpu.VMEM((_B, 64), jnp.float32),
            pltpu.VMEM((_B,), jnp.float32),
            pltpu.VMEM((_B,), jnp.float32),
            pltpu.VMEM((st,), jnp.float32),
            pltpu.VMEM_SHARED((n_pad,), jnp.float32),
            pltpu.VMEM_SHARED((n_pad, 64), jnp.float32),
            pltpu.SemaphoreType.DMA,
            pltpu.SemaphoreType.DMA,
            pltpu.SemaphoreType.DMA,
            pltpu.SemaphoreType.DMA,
            pltpu.SemaphoreType.DMA,
            pltpu.SemaphoreType.DMA,
            pltpu.SemaphoreType.DMA,
            pltpu.SemaphoreType.DMA,
            pltpu.SemaphoreType.DMA,
        ],
    )
    def k(row_hbm, col_hbm, w_hbm, p_hbm, s_out, dis_out,
          row2, col2, w2, wn_v, rows_0, rows_1,
          sc_0, sc_1, sc_2, sc_3, db_0, db_1, stripe_v,
          deg_sh, s_sh,
          sem_deg, sg0, sg1, ss0, ss1, ss2, ss3, sd0, sd1):
        c = lax.axis_index("c")
        s = lax.axis_index("s")
        zero_v = sc_0

        def zb(i, carry):
            for jj in range(4):
                zero_v[i, pl.ds(jj * 16, 16)] = jnp.zeros((16,), jnp.float32)
            return carry
        lax.fori_loop(0, _B, zb, 0)

        def zs(i, carry):
            stripe_v[pl.ds(i * 16, 16)] = jnp.zeros((16,), jnp.float32)
            return carry
        lax.fori_loop(0, st // 16, zs, 0)
        pltpu.sync_copy(stripe_v, deg_sh.at[pl.ds(s * st, st)])
        for kcp in range(ncp):
            pltpu.sync_copy(zero_v, s_sh.at[pl.ds(s * st + kcp * _B, _B)])
        plsc.subcore_barrier()

        def deg_chunk(start):
            pltpu.sync_copy(row_hbm.at[pl.ds(start, kb)], row2.at[pl.ds(0, kb)])
            pltpu.sync_copy(col_hbm.at[pl.ds(start, kb)], col2.at[pl.ds(0, kb)])
            pltpu.sync_copy(w_hbm.at[pl.ds(start, kb)], w2.at[pl.ds(0, kb)])

            def blk(j, carry):
                for i in range(_B // 16):
                    sl = pl.ds(i * 16, 16)
                    w2[j, sl] = jnp.where(row2[j, sl] == col2[j, sl], 0.0,
                                          w2[j, sl])
                pltpu.async_copy(w2.at[j], deg_sh.at[row2.at[j]], sem_deg,
                                 add=True)

                @pl.when(j >= 8)
                def _():
                    pltpu.make_async_copy(w2.at[0], deg_sh.at[row2.at[0]],
                                          sem_deg).wait()
                return carry
            lax.fori_loop(0, kb, blk, 0)

            def drain(j, carry):
                pltpu.make_async_copy(w2.at[0], deg_sh.at[row2.at[0]],
                                      sem_deg).wait()
                return carry
            lax.fori_loop(0, min(kb, 8), drain, 0)

        deg_chunk(s * 2 * kb)
        deg_chunk(s * 2 * kb + kb)
        plsc.subcore_barrier()

        pltpu.sync_copy(deg_sh.at[pl.ds(s * st, st)], stripe_v)

        def disb(i, carry):
            sl = pl.ds(i * 16, 16)
            stripe_v[sl] = _newton_rsqrt(stripe_v[sl])
            return carry
        lax.fori_loop(0, st // 16, disb, 0)
        pltpu.sync_copy(stripe_v, dis_out.at[c, pl.ds(s * st, st)])
        plsc.subcore_barrier()

        rbufs = (rows_0, rows_1)
        sbufs = (sc_0, sc_1, sc_2, sc_3)
        dbufs = (db_0, db_1)
        sgs = (sg0, sg1)
        sss = (ss0, ss1, ss2, ss3)
        sds = (sd0, sd1)

        def edge_phase(start, cnt):
            pltpu.sync_copy(row_hbm.at[pl.ds(start, cnt)],
                            row2.at[pl.ds(0, cnt)])
            pltpu.sync_copy(col_hbm.at[pl.ds(start, cnt)],
                            col2.at[pl.ds(0, cnt)])
            pltpu.sync_copy(w_hbm.at[pl.ds(start, cnt)],
                            w2.at[pl.ds(0, cnt)])
            pltpu.async_copy(p_hbm.at[row2.at[0]], rbufs[0], sgs[0])
            pltpu.async_copy(dis_out.at[c].at[row2.at[0]], dbufs[0], sds[0])

            def rnd(g, carry):
                for b in range(4):
                    j = g * 4 + b
                    b2 = b % 2
                    nb2 = 1 - b2

                    @pl.when(j + 1 < cnt)
                    def _():
                        pltpu.async_copy(p_hbm.at[row2.at[j + 1]], rbufs[nb2],
                                         sgs[nb2])
                        pltpu.async_copy(dis_out.at[c].at[row2.at[j + 1]],
                                         dbufs[nb2], sds[nb2])
                    pltpu.make_async_copy(p_hbm.at[row2.at[j]], rbufs[b2],
                                          sgs[b2]).wait()
                    pltpu.make_async_copy(dis_out.at[c].at[row2.at[j]],
                                          dbufs[b2], sds[b2]).wait()
                    for i in range(_B // 16):
                        sl = pl.ds(i * 16, 16)
                        wn_v[sl] = jnp.where(
                            row2[j, sl] == col2[j, sl], 0.0,
                            -w2[j, sl]) * dbufs[b2][sl]

                    @pl.when(j >= 4)
                    def _():
                        pltpu.make_async_copy(sbufs[b],
                                              s_sh.at[col2.at[j - 4]],
                                              sss[b]).wait()

                    def scale(g2, carry2):
                        chunk = wn_v[pl.ds(g2 * 16, 16)]
                        base = g2 * 16
                        for k2 in range(16):
                            f = chunk[k2]
                            for jj in range(4):
                                sl2 = pl.ds(jj * 16, 16)
                                sbufs[b][base + k2, sl2] = (
                                    rbufs[b2][base + k2, sl2] * f)
                        return carry2
                    lax.fori_loop(0, _B // 16, scale, 0)
                    pltpu.async_copy(sbufs[b], s_sh.at[col2.at[j]], sss[b],
                                     add=True)
                return carry
            lax.fori_loop(0, cnt // 4, rnd, 0)
            for b in range(4):
                pltpu.make_async_copy(sbufs[b], s_sh.at[col2.at[cnt - 4 + b]],
                                      sss[b]).wait()

        @pl.when(c == 0)
        def _():
            edge_phase(s * kb0, kb0)

        @pl.when(c == 1)
        def _():
            edge_phase(16 * kb0 + s * kb1, kb1)
        plsc.subcore_barrier()
        pltpu.sync_copy(s_sh.at[pl.ds(s * st, st)],
                        s_out.at[c, pl.ds(s * st, st)])

    return k


def _tc_dense(xs, wcat, bn):
    n, f = xs.shape

    def body(xs_ref, w_ref, o_ref):
        o_ref[...] = jnp.dot(xs_ref[...], w_ref[...],
                             preferred_element_type=jnp.float32)

    return pl.pallas_call(
        body,
        grid=(n // bn,),
        in_specs=[
            pl.BlockSpec((bn, f), lambda i: (i, 0)),
            pl.BlockSpec((f, 128), lambda i: (0, 0)),
        ],
        out_specs=pl.BlockSpec((bn, 128), lambda i: (i, 0)),
        out_shape=jax.ShapeDtypeStruct((n, 128), jnp.float32),
    )(xs, wcat)


def _tc_gates(d, s2, disT, bz, bh, wlin, blin, bn):
    n = d.shape[0]
    hor = wlin.shape[1]

    def body(d_ref, s_ref, dis_ref, bz_ref, bh_ref, wl_ref, bl_ref, o_ref):
        dis = dis_ref[:, 0:1]
        sv = (s_ref[0] + s_ref[1]) * dis
        a = d_ref[:, 64:128] + sv
        z = jax.nn.sigmoid(a[:, :32] + bz_ref[...])
        ht = jnp.tanh(a[:, 32:] + bh_ref[...])
        h = jnp.maximum((1.0 - z) * ht, 0.0)
        o_ref[...] = jnp.dot(h, wl_ref[...],
                             preferred_element_type=jnp.float32) + bl_ref[...]

    return pl.pallas_call(
        body,
        grid=(n // bn,),
        in_specs=[
            pl.BlockSpec((bn, 128), lambda i: (i, 0)),
            pl.BlockSpec((2, bn, 64), lambda i: (0, i, 0)),
            pl.BlockSpec((bn, 2), lambda i: (i, 0)),
            pl.BlockSpec((1, 32), lambda i: (0, 0)),
            pl.BlockSpec((1, 32), lambda i: (0, 0)),
            pl.BlockSpec((32, hor), lambda i: (0, 0)),
            pl.BlockSpec((1, hor), lambda i: (0, 0)),
        ],
        out_specs=pl.BlockSpec((bn, hor), lambda i: (i, 0)),
        out_shape=jax.ShapeDtypeStruct((n, hor), jnp.float32),
    )(d, s2, disT, bz, bh, wlin, blin)


def kernel(x, edge_index, edge_weight, Wxz0, Wxz1, bxz, Whz0, Whz1, bhz,
           Wxr0, Wxr1, bxr, Whr0, Whr1, bhr, Wxh0, Wxh1, bxh,
           Whh0, Whh1, bhh, Wlin, blin):
    xs = jnp.squeeze(x, 1)
    n = xs.shape[0]
    e = edge_index.shape[1]

    kb = -(-e // (_NW * _B))
    kb += (-kb) % 4
    ep = _NW * kb * _B
    n_pad = -(-n // 2048) * 2048

    row3 = jnp.pad(edge_index[0], (0, ep - e)).reshape(_NW * kb, _B)
    col3 = jnp.pad(edge_index[1], (0, ep - e)).reshape(_NW * kb, _B)
    w3 = jnp.pad(edge_weight, (0, ep - e)).reshape(_NW * kb, _B)

    wcat = jnp.concatenate([Wxz1, Wxh1, Wxz0, Wxh0], axis=1)
    bz = (bxz + bhz).reshape(1, -1)
    bh = (bxh + bhh).reshape(1, -1)
    blin2 = blin.reshape(1, -1)

    kb0 = (6 * kb // 5) & ~3
    kb1 = 2 * kb - kb0

    bn = 400
    o = _tc_dense(xs, wcat, bn)
    s2, dis2 = _sc_kernel(n_pad, kb, kb0, kb1)(row3, col3, w3, o)
    disT = dis2[0:2].T[:n]
    return _tc_gates(o, s2, disT, bz, bh, Wlin, blin2, bn)


# speedup vs baseline: 1.3850x; 1.3850x over previous
import functools

import jax
import jax.numpy as jnp
from jax import lax
from jax.experimental import pallas as pl
from jax.experimental.pallas import tpu as pltpu
from jax.experimental.pallas import tpu_sc as plsc

_NW = 32
_B = 128


def _newton_rsqrt(d):
    i = lax.bitcast_convert_type(d, jnp.int32)
    i = jnp.int32(0x5F3759DF) - (i >> 1)
    y = lax.bitcast_convert_type(i, jnp.float32)
    for _ in range(3):
        y = y * (1.5 - 0.5 * d * y * y)
    return jnp.where(d > 0, y, 0.0)


def _sc_kernel(n_pad, kb, kb0, kb1):
    st = n_pad // 16
    ncp = st // _B
    kbm = max(kb0, kb1)
    mesh = plsc.VectorSubcoreMesh(core_axis_name="c", subcore_axis_name="s")

    @functools.partial(
        pl.kernel, mesh=mesh,
        compiler_params=pltpu.CompilerParams(use_tc_tiling_on_sc=False),
        out_type=[
            jax.ShapeDtypeStruct((2, n_pad, 64), jnp.float32),
            jax.ShapeDtypeStruct((8, n_pad), jnp.float32),
        ],
        scratch_types=[
            pltpu.VMEM((kbm, _B), jnp.int32),
            pltpu.VMEM((kbm, _B), jnp.int32),
            pltpu.VMEM((kbm, _B), jnp.float32),
            pltpu.VMEM((kbm, _B), jnp.float32),
            pltpu.VMEM((_B,), jnp.float32),
            pltpu.VMEM((_B, 64), jnp.float32),
            pltpu.VMEM((_B, 64), jnp.float32),
            pltpu.VMEM((_B, 64), jnp.float32),
            pltpu.VMEM((_B, 64), jnp.float32),
            pltpu.VMEM((_B,), jnp.float32),
            pltpu.VMEM((_B,), jnp.float32),
            pltpu.VMEM((_B,), jnp.float32),
            pltpu.VMEM((_B,), jnp.float32),
            pltpu.VMEM((_B, 64), jnp.float32),
            pltpu.VMEM((st,), jnp.float32),
            pltpu.VMEM_SHARED((n_pad,), jnp.float32),
            pltpu.VMEM_SHARED((n_pad, 64), jnp.float32),
            pltpu.SemaphoreType.DMA,
            pltpu.SemaphoreType.DMA,
            pltpu.SemaphoreType.DMA,
            pltpu.SemaphoreType.DMA,
            pltpu.SemaphoreType.DMA,
            pltpu.SemaphoreType.DMA,
            pltpu.SemaphoreType.DMA,
            pltpu.SemaphoreType.DMA,
            pltpu.SemaphoreType.DMA,
            pltpu.SemaphoreType.DMA,
            pltpu.SemaphoreType.DMA,
            pltpu.SemaphoreType.DMA,
            pltpu.SemaphoreType.DMA,
        ],
    )
    def k(row_hbm, col_hbm, w_hbm, p_hbm, s_out, dis_out,
          row2, col2, w2, wz2, wn_v, rows_0, rows_1, rows_2, rows_3,
          db_0, db_1, db_2, db_3, zero_v, stripe_v, deg_sh, s_sh,
          sem_deg, sg0, sg1, sg2, sg3, ss0, ss1, ss2, ss3,
          sd0, sd1, sd2, sd3):
        c = lax.axis_index("c")
        s = lax.axis_index("s")

        def zb(i, carry):
            for jj in range(4):
                zero_v[i, pl.ds(jj * 16, 16)] = jnp.zeros((16,), jnp.float32)
            return carry
        lax.fori_loop(0, _B, zb, 0)

        def zs(i, carry):
            stripe_v[pl.ds(i * 16, 16)] = jnp.zeros((16,), jnp.float32)
            return carry
        lax.fori_loop(0, st // 16, zs, 0)
        pltpu.sync_copy(stripe_v, deg_sh.at[pl.ds(s * st, st)])
        for kcp in range(ncp):
            pltpu.sync_copy(zero_v, s_sh.at[pl.ds(s * st + kcp * _B, _B)])
        plsc.subcore_barrier()

        def deg_chunk(start):
            pltpu.sync_copy(row_hbm.at[pl.ds(start, kb)], row2.at[pl.ds(0, kb)])
            pltpu.sync_copy(col_hbm.at[pl.ds(start, kb)], col2.at[pl.ds(0, kb)])
            pltpu.sync_copy(w_hbm.at[pl.ds(start, kb)], w2.at[pl.ds(0, kb)])

            def blk(j, carry):
                for i in range(_B // 16):
                    sl = pl.ds(i * 16, 16)
                    wz2[j, sl] = jnp.where(row2[j, sl] == col2[j, sl], 0.0,
                                           w2[j, sl])
                pltpu.async_copy(wz2.at[j], deg_sh.at[row2.at[j]], sem_deg,
                                 add=True)

                @pl.when(j >= 8)
                def _():
                    pltpu.make_async_copy(wz2.at[0], deg_sh.at[row2.at[0]],
                                          sem_deg).wait()
                return carry
            lax.fori_loop(0, kb, blk, 0)

            def drain(j, carry):
                pltpu.make_async_copy(wz2.at[0], deg_sh.at[row2.at[0]],
                                      sem_deg).wait()
                return carry
            lax.fori_loop(0, min(kb, 8), drain, 0)

        deg_chunk(s * 2 * kb)
        deg_chunk(s * 2 * kb + kb)
        plsc.subcore_barrier()

        pltpu.sync_copy(deg_sh.at[pl.ds(s * st, st)], stripe_v)

        def disb(i, carry):
            sl = pl.ds(i * 16, 16)
            stripe_v[sl] = _newton_rsqrt(stripe_v[sl])
            return carry
        lax.fori_loop(0, st // 16, disb, 0)
        pltpu.sync_copy(stripe_v, dis_out.at[c, pl.ds(s * st, st)])
        plsc.subcore_barrier()

        rbufs = (rows_0, rows_1, rows_2, rows_3)
        dbufs = (db_0, db_1, db_2, db_3)
        sgs = (sg0, sg1, sg2, sg3)
        sss = (ss0, ss1, ss2, ss3)
        sds = (sd0, sd1, sd2, sd3)

        def edge_phase(start, cnt):
            pltpu.sync_copy(row_hbm.at[pl.ds(start, cnt)],
                            row2.at[pl.ds(0, cnt)])
            pltpu.sync_copy(col_hbm.at[pl.ds(start, cnt)],
                            col2.at[pl.ds(0, cnt)])
            pltpu.sync_copy(w_hbm.at[pl.ds(start, cnt)],
                            w2.at[pl.ds(0, cnt)])
            for j0 in range(2):
                pltpu.async_copy(p_hbm.at[row2.at[j0]], rbufs[j0], sgs[j0])
                pltpu.async_copy(dis_out.at[c].at[row2.at[j0]], dbufs[j0],
                                 sds[j0])

            def rnd(g, carry):
                for b in range(4):
                    j = g * 4 + b
                    pltpu.make_async_copy(p_hbm.at[row2.at[j]], rbufs[b],
                                          sgs[b]).wait()
                    pltpu.make_async_copy(dis_out.at[c].at[row2.at[j]],
                                          dbufs[b], sds[b]).wait()
                    for i in range(_B // 16):
                        sl = pl.ds(i * 16, 16)
                        wn_v[sl] = jnp.where(
                            row2[j, sl] == col2[j, sl], 0.0,
                            -w2[j, sl]) * dbufs[b][sl]

                    def scale(g2, carry2):
                        chunk = wn_v[pl.ds(g2 * 16, 16)]
                        base = g2 * 16
                        for k2 in range(16):
                            f = chunk[k2]
                            for jj in range(4):
                                sl2 = pl.ds(jj * 16, 16)
                                rbufs[b][base + k2, sl2] = (
                                    rbufs[b][base + k2, sl2] * f)
                        return carry2
                    lax.fori_loop(0, _B // 16, scale, 0)

                    nb = (b + 2) % 4

                    @pl.when(j >= 2)
                    def _():
                        pltpu.make_async_copy(rbufs[nb],
                                              s_sh.at[col2.at[j - 2]],
                                              sss[nb]).wait()
                    pltpu.async_copy(rbufs[b], s_sh.at[col2.at[j]], sss[b],
                                     add=True)

                    @pl.when(j + 2 < cnt)
                    def _():
                        pltpu.async_copy(p_hbm.at[row2.at[j + 2]], rbufs[nb],
                                         sgs[nb])
                        pltpu.async_copy(dis_out.at[c].at[row2.at[j + 2]],
                                         dbufs[nb], sds[nb])
                return carry
            lax.fori_loop(0, cnt // 4, rnd, 0)
            pltpu.make_async_copy(rbufs[2], s_sh.at[col2.at[cnt - 2]],
                                  sss[2]).wait()
            pltpu.make_async_copy(rbufs[3], s_sh.at[col2.at[cnt - 1]],
                                  sss[3]).wait()

        @pl.when(c == 0)
        def _():
            edge_phase(s * kb0, kb0)

        @pl.when(c == 1)
        def _():
            edge_phase(16 * kb0 + s * kb1, kb1)
        plsc.subcore_barrier()
        pltpu.sync_copy(s_sh.at[pl.ds(s * st, st)],
                        s_out.at[c, pl.ds(s * st, st)])

    return k


def _tc_dense(xs, wc0, wc1, bn):
    n, f = xs.shape

    def body(xs_ref, w0_ref, w1_ref, d_ref, p_ref):
        xv = xs_ref[...]
        d_ref[...] = jnp.dot(xv, w0_ref[...], preferred_element_type=jnp.float32)
        p_ref[...] = jnp.dot(xv, w1_ref[...], preferred_element_type=jnp.float32)

    return pl.pallas_call(
        body,
        grid=(n // bn,),
        in_specs=[
            pl.BlockSpec((bn, f), lambda i: (i, 0)),
            pl.BlockSpec((f, 64), lambda i: (0, 0)),
            pl.BlockSpec((f, 64), lambda i: (0, 0)),
        ],
        out_specs=[
            pl.BlockSpec((bn, 64), lambda i: (i, 0)),
            pl.BlockSpec((bn, 64), lambda i: (i, 0)),
        ],
        out_shape=[
            jax.ShapeDtypeStruct((n, 64), jnp.float32),
            jax.ShapeDtypeStruct((n, 64), jnp.float32),
        ],
    )(xs, wc0, wc1)


def _tc_gates(d, s2, disT, bz, bh, wlin, blin, bn):
    n = d.shape[0]
    hor = wlin.shape[1]

    def body(d_ref, s_ref, dis_ref, bz_ref, bh_ref, wl_ref, bl_ref, o_ref):
        dis = dis_ref[:, 0:1]
        sv = (s_ref[0] + s_ref[1]) * dis
        a = d_ref[...] + sv
        z = jax.nn.sigmoid(a[:, :32] + bz_ref[...])
        ht = jnp.tanh(a[:, 32:] + bh_ref[...])
        h = jnp.maximum((1.0 - z) * ht, 0.0)
        o_ref[...] = jnp.dot(h, wl_ref[...],
                             preferred_element_type=jnp.float32) + bl_ref[...]

    return pl.pallas_call(
        body,
        grid=(n // bn,),
        in_specs=[
            pl.BlockSpec((bn, 64), lambda i: (i, 0)),
            pl.BlockSpec((2, bn, 64), lambda i: (0, i, 0)),
            pl.BlockSpec((bn, 2), lambda i: (i, 0)),
            pl.BlockSpec((1, 32), lambda i: (0, 0)),
            pl.BlockSpec((1, 32), lambda i: (0, 0)),
            pl.BlockSpec((32, hor), lambda i: (0, 0)),
            pl.BlockSpec((1, hor), lambda i: (0, 0)),
        ],
        out_specs=pl.BlockSpec((bn, hor), lambda i: (i, 0)),
        out_shape=jax.ShapeDtypeStruct((n, hor), jnp.float32),
    )(d, s2, disT, bz, bh, wlin, blin)


def kernel(x, edge_index, edge_weight, Wxz0, Wxz1, bxz, Whz0, Whz1, bhz,
           Wxr0, Wxr1, bxr, Whr0, Whr1, bhr, Wxh0, Wxh1, bxh,
           Whh0, Whh1, bhh, Wlin, blin):
    xs = jnp.squeeze(x, 1)
    n = xs.shape[0]
    e = edge_index.shape[1]

    kb = -(-e // (_NW * _B))
    kb += (-kb) % 4
    ep = _NW * kb * _B
    n_pad = -(-n // 2048) * 2048

    row3 = jnp.pad(edge_index[0], (0, ep - e)).reshape(_NW * kb, _B)
    col3 = jnp.pad(edge_index[1], (0, ep - e)).reshape(_NW * kb, _B)
    w3 = jnp.pad(edge_weight, (0, ep - e)).reshape(_NW * kb, _B)

    wc0 = jnp.concatenate([Wxz0, Wxh0], axis=1)
    wc1 = jnp.concatenate([Wxz1, Wxh1], axis=1)
    bz = (bxz + bhz).reshape(1, -1)
    bh = (bxh + bhh).reshape(1, -1)
    blin2 = blin.reshape(1, -1)

    kb0 = (6 * kb // 5) & ~3
    kb1 = 2 * kb - kb0

    bn = 400
    d, p = _tc_dense(xs, wc0, wc1, bn)
    s2, dis2 = _sc_kernel(n_pad, kb, kb0, kb1)(row3, col3, w3, p)
    disT = dis2[0:2].T[:n]
    return _tc_gates(d, s2, disT, bz, bh, Wlin, blin2, bn)


# speedup vs baseline: 1.3907x; 1.0041x over previous
import functools

import jax
import jax.numpy as jnp
from jax import lax
from jax.experimental import pallas as pl
from jax.experimental.pallas import tpu as pltpu
from jax.experimental.pallas import tpu_sc as plsc

_NW = 32
_B = 128


def _newton_rsqrt(d):
    i = lax.bitcast_convert_type(d, jnp.int32)
    i = jnp.int32(0x5F3759DF) - (i >> 1)
    y = lax.bitcast_convert_type(i, jnp.float32)
    for _ in range(3):
        y = y * (1.5 - 0.5 * d * y * y)
    return jnp.where(d > 0, y, 0.0)


def _sc_kernel(n_pad, kb, kb0, kb1):
    st = n_pad // 16
    ncp = st // _B
    kbm = max(kb0, kb1)
    mesh = plsc.VectorSubcoreMesh(core_axis_name="c", subcore_axis_name="s")

    @functools.partial(
        pl.kernel, mesh=mesh,
        compiler_params=pltpu.CompilerParams(use_tc_tiling_on_sc=False),
        out_type=[
            jax.ShapeDtypeStruct((2, n_pad, 64), jnp.float32),
            jax.ShapeDtypeStruct((8, n_pad), jnp.float32),
        ],
        scratch_types=[
            pltpu.VMEM((kbm, _B), jnp.int32),
            pltpu.VMEM((kbm, _B), jnp.int32),
            pltpu.VMEM((kbm, _B), jnp.float32),
            pltpu.VMEM((kbm, _B), jnp.float32),
            pltpu.VMEM((_B,), jnp.float32),
            pltpu.VMEM((_B, 64), jnp.float32),
            pltpu.VMEM((_B, 64), jnp.float32),
            pltpu.VMEM((_B, 64), jnp.float32),
            pltpu.VMEM((_B, 64), jnp.float32),
            pltpu.VMEM((_B,), jnp.float32),
            pltpu.VMEM((_B,), jnp.float32),
            pltpu.VMEM((_B,), jnp.float32),
            pltpu.VMEM((_B,), jnp.float32),
            pltpu.VMEM((_B, 64), jnp.float32),
            pltpu.VMEM((st,), jnp.float32),
            pltpu.VMEM_SHARED((n_pad,), jnp.float32),
            pltpu.VMEM_SHARED((n_pad, 64), jnp.float32),
            pltpu.SemaphoreType.DMA,
            pltpu.SemaphoreType.DMA,
            pltpu.SemaphoreType.DMA,
            pltpu.SemaphoreType.DMA,
            pltpu.SemaphoreType.DMA,
            pltpu.SemaphoreType.DMA,
            pltpu.SemaphoreType.DMA,
            pltpu.SemaphoreType.DMA,
            pltpu.SemaphoreType.DMA,
            pltpu.SemaphoreType.DMA,
            pltpu.SemaphoreType.DMA,
            pltpu.SemaphoreType.DMA,
            pltpu.SemaphoreType.DMA,
        ],
    )
    def k(row_hbm, col_hbm, w_hbm, p_hbm, s_out, dis_out,
          row2, col2, w2, wz2, wn_v, rows_0, rows_1, rows_2, rows_3,
          db_0, db_1, db_2, db_3, zero_v, stripe_v, deg_sh, s_sh,
          sem_deg, sg0, sg1, sg2, sg3, ss0, ss1, ss2, ss3,
          sd0, sd1, sd2, sd3):
        c = lax.axis_index("c")
        s = lax.axis_index("s")

        def zb(i, carry):
            for jj in range(4):
                zero_v[i, pl.ds(jj * 16, 16)] = jnp.zeros((16,), jnp.float32)
            return carry
        lax.fori_loop(0, _B, zb, 0)

        def zs(i, carry):
            stripe_v[pl.ds(i * 16, 16)] = jnp.zeros((16,), jnp.float32)
            return carry
        lax.fori_loop(0, st // 16, zs, 0)
        pltpu.sync_copy(stripe_v, deg_sh.at[pl.ds(s * st, st)])
        for kcp in range(ncp):
            pltpu.sync_copy(zero_v, s_sh.at[pl.ds(s * st + kcp * _B, _B)])
        plsc.subcore_barrier()

        def deg_chunk(start):
            pltpu.sync_copy(row_hbm.at[pl.ds(start, kb)], row2.at[pl.ds(0, kb)])
            pltpu.sync_copy(col_hbm.at[pl.ds(start, kb)], col2.at[pl.ds(0, kb)])
            pltpu.sync_copy(w_hbm.at[pl.ds(start, kb)], w2.at[pl.ds(0, kb)])

            def blk(j, carry):
                for i in range(_B // 16):
                    sl = pl.ds(i * 16, 16)
                    wz2[j, sl] = jnp.where(row2[j, sl] == col2[j, sl], 0.0,
                                           w2[j, sl])
                pltpu.async_copy(wz2.at[j], deg_sh.at[row2.at[j]], sem_deg,
                                 add=True)

                @pl.when(j >= 8)
                def _():
                    pltpu.make_async_copy(wz2.at[0], deg_sh.at[row2.at[0]],
                                          sem_deg).wait()
                return carry
            lax.fori_loop(0, kb, blk, 0)

            def drain(j, carry):
                pltpu.make_async_copy(wz2.at[0], deg_sh.at[row2.at[0]],
                                      sem_deg).wait()
                return carry
            lax.fori_loop(0, min(kb, 8), drain, 0)

        deg_chunk(s * 2 * kb)
        deg_chunk(s * 2 * kb + kb)
        plsc.subcore_barrier()

        pltpu.sync_copy(deg_sh.at[pl.ds(s * st, st)], stripe_v)

        def disb(i, carry):
            sl = pl.ds(i * 16, 16)
            stripe_v[sl] = _newton_rsqrt(stripe_v[sl])
            return carry
        lax.fori_loop(0, st // 16, disb, 0)
        pltpu.sync_copy(stripe_v, dis_out.at[c, pl.ds(s * st, st)])
        plsc.subcore_barrier()

        rbufs = (rows_0, rows_1, rows_2, rows_3)
        dbufs = (db_0, db_1, db_2, db_3)
        sgs = (sg0, sg1, sg2, sg3)
        sss = (ss0, ss1, ss2, ss3)
        sds = (sd0, sd1, sd2, sd3)

        def edge_phase(start, cnt):
            pltpu.sync_copy(row_hbm.at[pl.ds(start, cnt)],
                            row2.at[pl.ds(0, cnt)])
            pltpu.sync_copy(col_hbm.at[pl.ds(start, cnt)],
                            col2.at[pl.ds(0, cnt)])
            pltpu.sync_copy(w_hbm.at[pl.ds(start, cnt)],
                            w2.at[pl.ds(0, cnt)])
            for j0 in range(2):
                pltpu.async_copy(p_hbm.at[row2.at[j0]], rbufs[j0], sgs[j0])
                pltpu.async_copy(dis_out.at[c].at[row2.at[j0]], dbufs[j0],
                                 sds[j0])

            def rnd(g, carry):
                for b in range(4):
                    j = g * 4 + b
                    pltpu.make_async_copy(p_hbm.at[row2.at[j]], rbufs[b],
                                          sgs[b]).wait()
                    pltpu.make_async_copy(dis_out.at[c].at[row2.at[j]],
                                          dbufs[b], sds[b]).wait()
                    for i in range(_B // 16):
                        sl = pl.ds(i * 16, 16)
                        wn_v[sl] = jnp.where(
                            row2[j, sl] == col2[j, sl], 0.0,
                            -w2[j, sl]) * dbufs[b][sl]

                    def scale(g2, carry2):
                        chunk = wn_v[pl.ds(g2 * 16, 16)]
                        base = g2 * 16
                        for k2 in range(16):
                            f = chunk[k2]
                            for jj in range(4):
                                sl2 = pl.ds(jj * 16, 16)
                                rbufs[b][base + k2, sl2] = (
                                    rbufs[b][base + k2, sl2] * f)
                        return carry2
                    lax.fori_loop(0, _B // 16, scale, 0)

                    nb = (b + 2) % 4

                    @pl.when(j >= 2)
                    def _():
                        pltpu.make_async_copy(rbufs[nb],
                                              s_sh.at[col2.at[j - 2]],
                                              sss[nb]).wait()
                    pltpu.async_copy(rbufs[b], s_sh.at[col2.at[j]], sss[b],
                                     add=True)

                    @pl.when(j + 2 < cnt)
                    def _():
                        pltpu.async_copy(p_hbm.at[row2.at[j + 2]], rbufs[nb],
                                         sgs[nb])
                        pltpu.async_copy(dis_out.at[c].at[row2.at[j + 2]],
                                         dbufs[nb], sds[nb])
                return carry
            lax.fori_loop(0, cnt // 4, rnd, 0)
            pltpu.make_async_copy(rbufs[2], s_sh.at[col2.at[cnt - 2]],
                                  sss[2]).wait()
            pltpu.make_async_copy(rbufs[3], s_sh.at[col2.at[cnt - 1]],
                                  sss[3]).wait()

        @pl.when(c == 0)
        def _():
            edge_phase(s * kb0, kb0)

        @pl.when(c == 1)
        def _():
            edge_phase(16 * kb0 + s * kb1, kb1)
        plsc.subcore_barrier()
        pltpu.sync_copy(s_sh.at[pl.ds(s * st, st)],
                        s_out.at[c, pl.ds(s * st, st)])

    return k


def _tc_dense(xs, wc0, wc1, bn):
    n, f = xs.shape

    def body(xs_ref, w0_ref, w1_ref, d_ref, p_ref):
        xv = xs_ref[...]
        d_ref[...] = jnp.dot(xv, w0_ref[...], preferred_element_type=jnp.float32)
        p_ref[...] = jnp.dot(xv, w1_ref[...], preferred_element_type=jnp.float32)

    return pl.pallas_call(
        body,
        grid=(n // bn,),
        in_specs=[
            pl.BlockSpec((bn, f), lambda i: (i, 0)),
            pl.BlockSpec((f, 64), lambda i: (0, 0)),
            pl.BlockSpec((f, 64), lambda i: (0, 0)),
        ],
        out_specs=[
            pl.BlockSpec((bn, 64), lambda i: (i, 0)),
            pl.BlockSpec((bn, 64), lambda i: (i, 0)),
        ],
        out_shape=[
            jax.ShapeDtypeStruct((n, 64), jnp.float32),
            jax.ShapeDtypeStruct((n, 64), jnp.float32),
        ],
    )(xs, wc0, wc1)


def _tc_gates(d, s2, disT, bz, bh, wlin, blin, bn):
    n = d.shape[0]
    hor = wlin.shape[1]

    def body(d_ref, s_ref, dis_ref, bz_ref, bh_ref, wl_ref, bl_ref, o_ref):
        dis = dis_ref[:, 0:1]
        sv = (s_ref[0] + s_ref[1]) * dis
        a = d_ref[...] + sv
        z = jax.nn.sigmoid(a[:, :32] + bz_ref[...])
        ht = jnp.tanh(a[:, 32:] + bh_ref[...])
        h = jnp.maximum((1.0 - z) * ht, 0.0)
        o_ref[...] = jnp.dot(h, wl_ref[...],
                             preferred_element_type=jnp.float32) + bl_ref[...]

    return pl.pallas_call(
        body,
        grid=(n // bn,),
        in_specs=[
            pl.BlockSpec((bn, 64), lambda i: (i, 0)),
            pl.BlockSpec((2, bn, 64), lambda i: (0, i, 0)),
            pl.BlockSpec((bn, 2), lambda i: (i, 0)),
            pl.BlockSpec((1, 32), lambda i: (0, 0)),
            pl.BlockSpec((1, 32), lambda i: (0, 0)),
            pl.BlockSpec((32, hor), lambda i: (0, 0)),
            pl.BlockSpec((1, hor), lambda i: (0, 0)),
        ],
        out_specs=pl.BlockSpec((bn, hor), lambda i: (i, 0)),
        out_shape=jax.ShapeDtypeStruct((n, hor), jnp.float32),
    )(d, s2, disT, bz, bh, wlin, blin)


def kernel(x, edge_index, edge_weight, Wxz0, Wxz1, bxz, Whz0, Whz1, bhz,
           Wxr0, Wxr1, bxr, Whr0, Whr1, bhr, Wxh0, Wxh1, bxh,
           Whh0, Whh1, bhh, Wlin, blin):
    xs = jnp.squeeze(x, 1)
    n = xs.shape[0]
    e = edge_index.shape[1]

    kb = -(-e // (_NW * _B))
    kb += (-kb) % 4
    ep = _NW * kb * _B
    n_pad = -(-n // 2048) * 2048

    row3 = jnp.pad(edge_index[0], (0, ep - e)).reshape(_NW * kb, _B)
    col3 = jnp.pad(edge_index[1], (0, ep - e)).reshape(_NW * kb, _B)
    w3 = jnp.pad(edge_weight, (0, ep - e)).reshape(_NW * kb, _B)

    wc0 = jnp.concatenate([Wxz0, Wxh0], axis=1)
    wc1 = jnp.concatenate([Wxz1, Wxh1], axis=1)
    bz = (bxz + bhz).reshape(1, -1)
    bh = (bxh + bhh).reshape(1, -1)
    blin2 = blin.reshape(1, -1)

    kb0 = (13 * kb // 10) & ~3
    kb1 = 2 * kb - kb0

    bn = 400
    d, p = _tc_dense(xs, wc0, wc1, bn)
    s2, dis2 = _sc_kernel(n_pad, kb, kb0, kb1)(row3, col3, w3, p)
    disT = dis2[0:2].T[:n]
    return _tc_gates(d, s2, disT, bz, bh, Wlin, blin2, bn)


# speedup vs baseline: 1.4955x; 1.0754x over previous
import functools

import jax
import jax.numpy as jnp
from jax import lax
from jax.experimental import pallas as pl
from jax.experimental.pallas import tpu as pltpu
from jax.experimental.pallas import tpu_sc as plsc

_NW = 32
_B = 128


def _newton_rsqrt(d):
    i = lax.bitcast_convert_type(d, jnp.int32)
    i = jnp.int32(0x5F3759DF) - (i >> 1)
    y = lax.bitcast_convert_type(i, jnp.float32)
    for _ in range(3):
        y = y * (1.5 - 0.5 * d * y * y)
    return jnp.where(d > 0, y, 0.0)


def _sc_deg_kernel(n_pad, kb):
    st = n_pad // 16
    mesh = plsc.VectorSubcoreMesh(core_axis_name="c", subcore_axis_name="s")

    @functools.partial(
        pl.kernel, mesh=mesh,
        compiler_params=pltpu.CompilerParams(use_tc_tiling_on_sc=False),
        out_type=jax.ShapeDtypeStruct((8, n_pad), jnp.float32),
        scratch_types=[
            pltpu.VMEM((kb, _B), jnp.int32),
            pltpu.VMEM((kb, _B), jnp.int32),
            pltpu.VMEM((kb, _B), jnp.float32),
            pltpu.VMEM((st,), jnp.float32),
            pltpu.VMEM_SHARED((n_pad,), jnp.float32),
            pltpu.SemaphoreType.DMA,
        ],
    )
    def k(row_hbm, col_hbm, w_hbm, dis_out, row2, col2, w2, stripe_v,
          deg_sh, sem_deg):
        c = lax.axis_index("c")
        s = lax.axis_index("s")

        def zs(i, carry):
            stripe_v[pl.ds(i * 16, 16)] = jnp.zeros((16,), jnp.float32)
            return carry
        lax.fori_loop(0, st // 16, zs, 0)
        pltpu.sync_copy(stripe_v, deg_sh.at[pl.ds(s * st, st)])
        plsc.subcore_barrier()

        def deg_chunk(start):
            pltpu.sync_copy(row_hbm.at[pl.ds(start, kb)], row2)
            pltpu.sync_copy(col_hbm.at[pl.ds(start, kb)], col2)
            pltpu.sync_copy(w_hbm.at[pl.ds(start, kb)], w2)

            def blk(j, carry):
                for i in range(_B // 16):
                    sl = pl.ds(i * 16, 16)
                    w2[j, sl] = jnp.where(row2[j, sl] == col2[j, sl], 0.0,
                                          w2[j, sl])
                pltpu.async_copy(w2.at[j], deg_sh.at[row2.at[j]], sem_deg,
                                 add=True)

                @pl.when(j >= 8)
                def _():
                    pltpu.make_async_copy(w2.at[0], deg_sh.at[row2.at[0]],
                                          sem_deg).wait()
                return carry
            lax.fori_loop(0, kb, blk, 0)

            def drain(j, carry):
                pltpu.make_async_copy(w2.at[0], deg_sh.at[row2.at[0]],
                                      sem_deg).wait()
                return carry
            lax.fori_loop(0, min(kb, 8), drain, 0)

        deg_chunk(s * 2 * kb)
        deg_chunk(s * 2 * kb + kb)
        plsc.subcore_barrier()

        pltpu.sync_copy(deg_sh.at[pl.ds(s * st, st)], stripe_v)

        def disb(i, carry):
            sl = pl.ds(i * 16, 16)
            stripe_v[sl] = _newton_rsqrt(stripe_v[sl])
            return carry
        lax.fori_loop(0, st // 16, disb, 0)
        pltpu.sync_copy(stripe_v, dis_out.at[c, pl.ds(s * st, st)])

    return k


def _sc_edge_kernel(n_pad, kb, kb0, kb1):
    st = n_pad // 16
    ncp = st // _B
    kbm = max(kb0, kb1)
    mesh = plsc.VectorSubcoreMesh(core_axis_name="c", subcore_axis_name="s")

    @functools.partial(
        pl.kernel, mesh=mesh,
        compiler_params=pltpu.CompilerParams(use_tc_tiling_on_sc=False),
        out_type=jax.ShapeDtypeStruct((2, n_pad, 64), jnp.float32),
        scratch_types=[
            pltpu.VMEM((kbm, _B), jnp.int32),
            pltpu.VMEM((kbm, _B), jnp.int32),
            pltpu.VMEM((kbm, _B), jnp.float32),
            pltpu.VMEM((_B,), jnp.float32),
            pltpu.VMEM((_B, 64), jnp.float32),
            pltpu.VMEM((_B, 64), jnp.float32),
            pltpu.VMEM((_B, 64), jnp.float32),
            pltpu.VMEM((_B, 64), jnp.float32),
            pltpu.VMEM((_B,), jnp.float32),
            pltpu.VMEM((_B,), jnp.float32),
            pltpu.VMEM((_B,), jnp.float32),
            pltpu.VMEM((_B,), jnp.float32),
            pltpu.VMEM((_B, 64), jnp.float32),
            pltpu.VMEM_SHARED((n_pad, 64), jnp.float32),
            pltpu.SemaphoreType.DMA,
            pltpu.SemaphoreType.DMA,
            pltpu.SemaphoreType.DMA,
            pltpu.SemaphoreType.DMA,
            pltpu.SemaphoreType.DMA,
            pltpu.SemaphoreType.DMA,
            pltpu.SemaphoreType.DMA,
            pltpu.SemaphoreType.DMA,
            pltpu.SemaphoreType.DMA,
            pltpu.SemaphoreType.DMA,
            pltpu.SemaphoreType.DMA,
            pltpu.SemaphoreType.DMA,
        ],
    )
    def k(row_hbm, col_hbm, w_hbm, p_hbm, dis_hbm, s_out,
          row2, col2, w2, wn_v, rows_0, rows_1, rows_2, rows_3,
          db_0, db_1, db_2, db_3, zero_v, s_sh,
          sg0, sg1, sg2, sg3, ss0, ss1, ss2, ss3,
          sd0, sd1, sd2, sd3):
        c = lax.axis_index("c")
        s = lax.axis_index("s")

        def zb(i, carry):
            for jj in range(4):
                zero_v[i, pl.ds(jj * 16, 16)] = jnp.zeros((16,), jnp.float32)
            return carry
        lax.fori_loop(0, _B, zb, 0)
        for kcp in range(ncp):
            pltpu.sync_copy(zero_v, s_sh.at[pl.ds(s * st + kcp * _B, _B)])
        plsc.subcore_barrier()

        rbufs = (rows_0, rows_1, rows_2, rows_3)
        dbufs = (db_0, db_1, db_2, db_3)
        sgs = (sg0, sg1, sg2, sg3)
        sss = (ss0, ss1, ss2, ss3)
        sds = (sd0, sd1, sd2, sd3)

        def edge_phase(start, cnt):
            pltpu.sync_copy(row_hbm.at[pl.ds(start, cnt)],
                            row2.at[pl.ds(0, cnt)])
            pltpu.sync_copy(col_hbm.at[pl.ds(start, cnt)],
                            col2.at[pl.ds(0, cnt)])
            pltpu.sync_copy(w_hbm.at[pl.ds(start, cnt)],
                            w2.at[pl.ds(0, cnt)])
            for j0 in range(2):
                pltpu.async_copy(p_hbm.at[row2.at[j0]], rbufs[j0], sgs[j0])
                pltpu.async_copy(dis_hbm.at[c].at[row2.at[j0]], dbufs[j0],
                                 sds[j0])

            def rnd(g, carry):
                for b in range(4):
                    j = g * 4 + b
                    pltpu.make_async_copy(p_hbm.at[row2.at[j]], rbufs[b],
                                          sgs[b]).wait()
                    pltpu.make_async_copy(dis_hbm.at[c].at[row2.at[j]],
                                          dbufs[b], sds[b]).wait()
                    for i in range(_B // 16):
                        sl = pl.ds(i * 16, 16)
                        wn_v[sl] = jnp.where(
                            row2[j, sl] == col2[j, sl], 0.0,
                            -w2[j, sl]) * dbufs[b][sl]

                    def scale(g2, carry2):
                        chunk = wn_v[pl.ds(g2 * 16, 16)]
                        base = g2 * 16
                        for k2 in range(16):
                            f = chunk[k2]
                            for jj in range(4):
                                sl2 = pl.ds(jj * 16, 16)
                                rbufs[b][base + k2, sl2] = (
                                    rbufs[b][base + k2, sl2] * f)
                        return carry2
                    lax.fori_loop(0, _B // 16, scale, 0)

                    nb = (b + 2) % 4

                    @pl.when(j >= 2)
                    def _():
                        pltpu.make_async_copy(rbufs[nb],
                                              s_sh.at[col2.at[j - 2]],
                                              sss[nb]).wait()
                    pltpu.async_copy(rbufs[b], s_sh.at[col2.at[j]], sss[b],
                                     add=True)

                    @pl.when(j + 2 < cnt)
                    def _():
                        pltpu.async_copy(p_hbm.at[row2.at[j + 2]], rbufs[nb],
                                         sgs[nb])
                        pltpu.async_copy(dis_hbm.at[c].at[row2.at[j + 2]],
                                         dbufs[nb], sds[nb])
                return carry
            lax.fori_loop(0, cnt // 4, rnd, 0)
            pltpu.make_async_copy(rbufs[2], s_sh.at[col2.at[cnt - 2]],
                                  sss[2]).wait()
            pltpu.make_async_copy(rbufs[3], s_sh.at[col2.at[cnt - 1]],
                                  sss[3]).wait()

        @pl.when(c == 0)
        def _():
            edge_phase(s * kb0, kb0)

        @pl.when(c == 1)
        def _():
            edge_phase(16 * kb0 + s * kb1, kb1)
        plsc.subcore_barrier()
        pltpu.sync_copy(s_sh.at[pl.ds(s * st, st)],
                        s_out.at[c, pl.ds(s * st, st)])

    return k


def _tc_dense(xs, wc0, wc1, bn):
    n, f = xs.shape

    def body(xs_ref, w0_ref, w1_ref, d_ref, p_ref):
        xv = xs_ref[...]
        d_ref[...] = jnp.dot(xv, w0_ref[...], preferred_element_type=jnp.float32)
        p_ref[...] = jnp.dot(xv, w1_ref[...], preferred_element_type=jnp.float32)

    return pl.pallas_call(
        body,
        grid=(n // bn,),
        in_specs=[
            pl.BlockSpec((bn, f), lambda i: (i, 0)),
            pl.BlockSpec((f, 64), lambda i: (0, 0)),
            pl.BlockSpec((f, 64), lambda i: (0, 0)),
        ],
        out_specs=[
            pl.BlockSpec((bn, 64), lambda i: (i, 0)),
            pl.BlockSpec((bn, 64), lambda i: (i, 0)),
        ],
        out_shape=[
            jax.ShapeDtypeStruct((n, 64), jnp.float32),
            jax.ShapeDtypeStruct((n, 64), jnp.float32),
        ],
    )(xs, wc0, wc1)


def _tc_gates(d, s2, disT, bz, bh, wlin, blin, bn):
    n = d.shape[0]
    hor = wlin.shape[1]

    def body(d_ref, s_ref, dis_ref, bz_ref, bh_ref, wl_ref, bl_ref, o_ref):
        dis = dis_ref[:, 0:1]
        sv = (s_ref[0] + s_ref[1]) * dis
        a = d_ref[...] + sv
        z = jax.nn.sigmoid(a[:, :32] + bz_ref[...])
        ht = jnp.tanh(a[:, 32:] + bh_ref[...])
        h = jnp.maximum((1.0 - z) * ht, 0.0)
        o_ref[...] = jnp.dot(h, wl_ref[...],
                             preferred_element_type=jnp.float32) + bl_ref[...]

    return pl.pallas_call(
        body,
        grid=(n // bn,),
        in_specs=[
            pl.BlockSpec((bn, 64), lambda i: (i, 0)),
            pl.BlockSpec((2, bn, 64), lambda i: (0, i, 0)),
            pl.BlockSpec((bn, 2), lambda i: (i, 0)),
            pl.BlockSpec((1, 32), lambda i: (0, 0)),
            pl.BlockSpec((1, 32), lambda i: (0, 0)),
            pl.BlockSpec((32, hor), lambda i: (0, 0)),
            pl.BlockSpec((1, hor), lambda i: (0, 0)),
        ],
        out_specs=pl.BlockSpec((bn, hor), lambda i: (i, 0)),
        out_shape=jax.ShapeDtypeStruct((n, hor), jnp.float32),
    )(d, s2, disT, bz, bh, wlin, blin)


def kernel(x, edge_index, edge_weight, Wxz0, Wxz1, bxz, Whz0, Whz1, bhz,
           Wxr0, Wxr1, bxr, Whr0, Whr1, bhr, Wxh0, Wxh1, bxh,
           Whh0, Whh1, bhh, Wlin, blin):
    xs = jnp.squeeze(x, 1)
    n = xs.shape[0]
    e = edge_index.shape[1]

    kb = -(-e // (_NW * _B))
    kb += (-kb) % 4
    ep = _NW * kb * _B
    n_pad = -(-n // 2048) * 2048

    row3 = jnp.pad(edge_index[0], (0, ep - e)).reshape(_NW * kb, _B)
    col3 = jnp.pad(edge_index[1], (0, ep - e)).reshape(_NW * kb, _B)
    w3 = jnp.pad(edge_weight, (0, ep - e)).reshape(_NW * kb, _B)

    wc0 = jnp.concatenate([Wxz0, Wxh0], axis=1)
    wc1 = jnp.concatenate([Wxz1, Wxh1], axis=1)
    bz = (bxz + bhz).reshape(1, -1)
    bh = (bxh + bhh).reshape(1, -1)
    blin2 = blin.reshape(1, -1)

    kb0 = (13 * kb // 10) & ~3
    kb1 = 2 * kb - kb0

    bn = 400
    d, p = _tc_dense(xs, wc0, wc1, bn)
    dis2 = _sc_deg_kernel(n_pad, kb)(row3, col3, w3)
    s2 = _sc_edge_kernel(n_pad, kb, kb0, kb1)(row3, col3, w3, p, dis2)
    disT = dis2[0:2].T[:n]
    return _tc_gates(d, s2, disT, bz, bh, Wlin, blin2, bn)


# speedup vs baseline: 1.4965x; 1.0007x over previous
import functools

import jax
import jax.numpy as jnp
from jax import lax
from jax.experimental import pallas as pl
from jax.experimental.pallas import tpu as pltpu
from jax.experimental.pallas import tpu_sc as plsc

_NW = 32
_B = 128


def _newton_rsqrt(d):
    i = lax.bitcast_convert_type(d, jnp.int32)
    i = jnp.int32(0x5F3759DF) - (i >> 1)
    y = lax.bitcast_convert_type(i, jnp.float32)
    for _ in range(3):
        y = y * (1.5 - 0.5 * d * y * y)
    return jnp.where(d > 0, y, 0.0)


def _sc_deg_kernel(n_pad, kb):
    st = n_pad // 16
    mesh = plsc.VectorSubcoreMesh(core_axis_name="c", subcore_axis_name="s")

    @functools.partial(
        pl.kernel, mesh=mesh,
        compiler_params=pltpu.CompilerParams(use_tc_tiling_on_sc=False),
        out_type=jax.ShapeDtypeStruct((8, n_pad), jnp.float32),
        scratch_types=[
            pltpu.VMEM((kb, _B), jnp.int32),
            pltpu.VMEM((kb, _B), jnp.int32),
            pltpu.VMEM((kb, _B), jnp.float32),
            pltpu.VMEM((st,), jnp.float32),
            pltpu.VMEM_SHARED((n_pad,), jnp.float32),
            pltpu.SemaphoreType.DMA,
        ],
    )
    def k(row_hbm, col_hbm, w_hbm, dis_out, row2, col2, w2, stripe_v,
          deg_sh, sem_deg):
        c = lax.axis_index("c")
        s = lax.axis_index("s")

        def zs(i, carry):
            stripe_v[pl.ds(i * 16, 16)] = jnp.zeros((16,), jnp.float32)
            return carry
        lax.fori_loop(0, st // 16, zs, 0)
        pltpu.sync_copy(stripe_v, deg_sh.at[pl.ds(s * st, st)])
        plsc.subcore_barrier()

        def deg_chunk(start):
            pltpu.sync_copy(row_hbm.at[pl.ds(start, kb)], row2)
            pltpu.sync_copy(col_hbm.at[pl.ds(start, kb)], col2)
            pltpu.sync_copy(w_hbm.at[pl.ds(start, kb)], w2)

            def blk(j, carry):
                for i in range(_B // 16):
                    sl = pl.ds(i * 16, 16)
                    w2[j, sl] = jnp.where(row2[j, sl] == col2[j, sl], 0.0,
                                          w2[j, sl])
                pltpu.async_copy(w2.at[j], deg_sh.at[row2.at[j]], sem_deg,
                                 add=True)

                @pl.when(j >= 8)
                def _():
                    pltpu.make_async_copy(w2.at[0], deg_sh.at[row2.at[0]],
                                          sem_deg).wait()
                return carry
            lax.fori_loop(0, kb, blk, 0)

            def drain(j, carry):
                pltpu.make_async_copy(w2.at[0], deg_sh.at[row2.at[0]],
                                      sem_deg).wait()
                return carry
            lax.fori_loop(0, min(kb, 8), drain, 0)

        deg_chunk(s * 2 * kb)
        deg_chunk(s * 2 * kb + kb)
        plsc.subcore_barrier()

        pltpu.sync_copy(deg_sh.at[pl.ds(s * st, st)], stripe_v)

        def disb(i, carry):
            sl = pl.ds(i * 16, 16)
            stripe_v[sl] = _newton_rsqrt(stripe_v[sl])
            return carry
        lax.fori_loop(0, st // 16, disb, 0)
        pltpu.sync_copy(stripe_v, dis_out.at[c, pl.ds(s * st, st)])

    return k


def _sc_edge_kernel(n_pad, kb, kb0, kb1):
    st = n_pad // 16
    ncp = st // _B
    kbm = max(kb0, kb1)
    mesh = plsc.VectorSubcoreMesh(core_axis_name="c", subcore_axis_name="s")

    @functools.partial(
        pl.kernel, mesh=mesh,
        compiler_params=pltpu.CompilerParams(use_tc_tiling_on_sc=False),
        out_type=jax.ShapeDtypeStruct((2, n_pad, 64), jnp.float32),
        scratch_types=[
            pltpu.VMEM((kbm, _B), jnp.int32),
            pltpu.VMEM((kbm, _B), jnp.int32),
            pltpu.VMEM((kbm, _B), jnp.float32),
            pltpu.VMEM((_B,), jnp.float32),
            pltpu.VMEM((_B, 64), jnp.float32),
            pltpu.VMEM((_B, 64), jnp.float32),
            pltpu.VMEM((_B, 64), jnp.float32),
            pltpu.VMEM((_B, 64), jnp.float32),
            pltpu.VMEM((_B,), jnp.float32),
            pltpu.VMEM((_B,), jnp.float32),
            pltpu.VMEM((_B,), jnp.float32),
            pltpu.VMEM((_B,), jnp.float32),
            pltpu.VMEM((_B, 64), jnp.float32),
            pltpu.VMEM_SHARED((n_pad, 64), jnp.float32),
            pltpu.SemaphoreType.DMA,
            pltpu.SemaphoreType.DMA,
            pltpu.SemaphoreType.DMA,
            pltpu.SemaphoreType.DMA,
            pltpu.SemaphoreType.DMA,
            pltpu.SemaphoreType.DMA,
            pltpu.SemaphoreType.DMA,
            pltpu.SemaphoreType.DMA,
            pltpu.SemaphoreType.DMA,
            pltpu.SemaphoreType.DMA,
            pltpu.SemaphoreType.DMA,
            pltpu.SemaphoreType.DMA,
        ],
    )
    def k(row_hbm, col_hbm, w_hbm, p_hbm, dis_hbm, s_out,
          row2, col2, w2, wn_v, rows_0, rows_1, rows_2, rows_3,
          db_0, db_1, db_2, db_3, zero_v, s_sh,
          sg0, sg1, sg2, sg3, ss0, ss1, ss2, ss3,
          sd0, sd1, sd2, sd3):
        c = lax.axis_index("c")
        s = lax.axis_index("s")

        def zb(i, carry):
            for jj in range(4):
                zero_v[i, pl.ds(jj * 16, 16)] = jnp.zeros((16,), jnp.float32)
            return carry
        lax.fori_loop(0, _B, zb, 0)
        for kcp in range(ncp):
            pltpu.async_copy(zero_v, s_sh.at[pl.ds(s * st + kcp * _B, _B)],
                             sg0)
        for kcp in range(ncp):
            pltpu.make_async_copy(zero_v, s_sh.at[pl.ds(s * st, _B)],
                                  sg0).wait()
        plsc.subcore_barrier()

        rbufs = (rows_0, rows_1, rows_2, rows_3)
        dbufs = (db_0, db_1, db_2, db_3)
        sgs = (sg0, sg1, sg2, sg3)
        sss = (ss0, ss1, ss2, ss3)
        sds = (sd0, sd1, sd2, sd3)

        def edge_phase(start, cnt):
            pltpu.sync_copy(row_hbm.at[pl.ds(start, cnt)],
                            row2.at[pl.ds(0, cnt)])
            pltpu.sync_copy(col_hbm.at[pl.ds(start, cnt)],
                            col2.at[pl.ds(0, cnt)])
            pltpu.sync_copy(w_hbm.at[pl.ds(start, cnt)],
                            w2.at[pl.ds(0, cnt)])
            for j0 in range(2):
                pltpu.async_copy(p_hbm.at[row2.at[j0]], rbufs[j0], sgs[j0])
                pltpu.async_copy(dis_hbm.at[c].at[row2.at[j0]], dbufs[j0],
                                 sds[j0])

            def rnd(g, carry):
                for b in range(4):
                    j = g * 4 + b
                    pltpu.make_async_copy(p_hbm.at[row2.at[j]], rbufs[b],
                                          sgs[b]).wait()
                    pltpu.make_async_copy(dis_hbm.at[c].at[row2.at[j]],
                                          dbufs[b], sds[b]).wait()
                    for i in range(_B // 16):
                        sl = pl.ds(i * 16, 16)
                        wn_v[sl] = jnp.where(
                            row2[j, sl] == col2[j, sl], 0.0,
                            -w2[j, sl]) * dbufs[b][sl]

                    def scale(g2, carry2):
                        chunk = wn_v[pl.ds(g2 * 16, 16)]
                        base = g2 * 16
                        for k2 in range(16):
                            f = chunk[k2]
                            for jj in range(4):
                                sl2 = pl.ds(jj * 16, 16)
                                rbufs[b][base + k2, sl2] = (
                                    rbufs[b][base + k2, sl2] * f)
                        return carry2
                    lax.fori_loop(0, _B // 16, scale, 0)

                    nb = (b + 2) % 4

                    @pl.when(j >= 2)
                    def _():
                        pltpu.make_async_copy(rbufs[nb],
                                              s_sh.at[col2.at[j - 2]],
                                              sss[nb]).wait()
                    pltpu.async_copy(rbufs[b], s_sh.at[col2.at[j]], sss[b],
                                     add=True)

                    @pl.when(j + 2 < cnt)
                    def _():
                        pltpu.async_copy(p_hbm.at[row2.at[j + 2]], rbufs[nb],
                                         sgs[nb])
                        pltpu.async_copy(dis_hbm.at[c].at[row2.at[j + 2]],
                                         dbufs[nb], sds[nb])
                return carry
            lax.fori_loop(0, cnt // 4, rnd, 0)
            pltpu.make_async_copy(rbufs[2], s_sh.at[col2.at[cnt - 2]],
                                  sss[2]).wait()
            pltpu.make_async_copy(rbufs[3], s_sh.at[col2.at[cnt - 1]],
                                  sss[3]).wait()

        @pl.when(c == 0)
        def _():
            edge_phase(s * kb0, kb0)

        @pl.when(c == 1)
        def _():
            edge_phase(16 * kb0 + s * kb1, kb1)
        plsc.subcore_barrier()
        pltpu.sync_copy(s_sh.at[pl.ds(s * st, st)],
                        s_out.at[c, pl.ds(s * st, st)])

    return k


def _tc_dense(xs, wc0, wc1, bn):
    n, f = xs.shape

    def body(xs_ref, w0_ref, w1_ref, d_ref, p_ref):
        xv = xs_ref[...]
        d_ref[...] = jnp.dot(xv, w0_ref[...], preferred_element_type=jnp.float32)
        p_ref[...] = jnp.dot(xv, w1_ref[...], preferred_element_type=jnp.float32)

    return pl.pallas_call(
        body,
        grid=(n // bn,),
        in_specs=[
            pl.BlockSpec((bn, f), lambda i: (i, 0)),
            pl.BlockSpec((f, 64), lambda i: (0, 0)),
            pl.BlockSpec((f, 64), lambda i: (0, 0)),
        ],
        out_specs=[
            pl.BlockSpec((bn, 64), lambda i: (i, 0)),
            pl.BlockSpec((bn, 64), lambda i: (i, 0)),
        ],
        out_shape=[
            jax.ShapeDtypeStruct((n, 64), jnp.float32),
            jax.ShapeDtypeStruct((n, 64), jnp.float32),
        ],
    )(xs, wc0, wc1)


def _tc_gates(d, s2, disT, bz, bh, wlin, blin, bn):
    n = d.shape[0]
    hor = wlin.shape[1]

    def body(d_ref, s_ref, dis_ref, bz_ref, bh_ref, wl_ref, bl_ref, o_ref):
        dis = dis_ref[:, 0:1]
        sv = (s_ref[0] + s_ref[1]) * dis
        a = d_ref[...] + sv
        z = jax.nn.sigmoid(a[:, :32] + bz_ref[...])
        ht = jnp.tanh(a[:, 32:] + bh_ref[...])
        h = jnp.maximum((1.0 - z) * ht, 0.0)
        o_ref[...] = jnp.dot(h, wl_ref[...],
                             preferred_element_type=jnp.float32) + bl_ref[...]

    return pl.pallas_call(
        body,
        grid=(n // bn,),
        in_specs=[
            pl.BlockSpec((bn, 64), lambda i: (i, 0)),
            pl.BlockSpec((2, bn, 64), lambda i: (0, i, 0)),
            pl.BlockSpec((bn, 2), lambda i: (i, 0)),
            pl.BlockSpec((1, 32), lambda i: (0, 0)),
            pl.BlockSpec((1, 32), lambda i: (0, 0)),
            pl.BlockSpec((32, hor), lambda i: (0, 0)),
            pl.BlockSpec((1, hor), lambda i: (0, 0)),
        ],
        out_specs=pl.BlockSpec((bn, hor), lambda i: (i, 0)),
        out_shape=jax.ShapeDtypeStruct((n, hor), jnp.float32),
    )(d, s2, disT, bz, bh, wlin, blin)


def kernel(x, edge_index, edge_weight, Wxz0, Wxz1, bxz, Whz0, Whz1, bhz,
           Wxr0, Wxr1, bxr, Whr0, Whr1, bhr, Wxh0, Wxh1, bxh,
           Whh0, Whh1, bhh, Wlin, blin):
    xs = jnp.squeeze(x, 1)
    n = xs.shape[0]
    e = edge_index.shape[1]

    kb = -(-e // (_NW * _B))
    kb += (-kb) % 4
    ep = _NW * kb * _B
    n_pad = -(-n // 2048) * 2048

    row3 = jnp.pad(edge_index[0], (0, ep - e)).reshape(_NW * kb, _B)
    col3 = jnp.pad(edge_index[1], (0, ep - e)).reshape(_NW * kb, _B)
    w3 = jnp.pad(edge_weight, (0, ep - e)).reshape(_NW * kb, _B)

    wc0 = jnp.concatenate([Wxz0, Wxh0], axis=1)
    wc1 = jnp.concatenate([Wxz1, Wxh1], axis=1)
    bz = (bxz + bhz).reshape(1, -1)
    bh = (bxh + bhh).reshape(1, -1)
    blin2 = blin.reshape(1, -1)

    kb0 = (13 * kb // 10) & ~3
    kb1 = 2 * kb - kb0

    bn = 400
    d, p = _tc_dense(xs, wc0, wc1, bn)
    dis2 = _sc_deg_kernel(n_pad, kb)(row3, col3, w3)
    s2 = _sc_edge_kernel(n_pad, kb, kb0, kb1)(row3, col3, w3, p, dis2)
    disT = dis2[0:2].T[:n]
    return _tc_gates(d, s2, disT, bz, bh, Wlin, blin2, bn)
